# Initial kernel scaffold; baseline (speedup 1.0000x reference)
import jax
import jax.numpy as jnp
from jax.experimental import pallas as pl


def kernel(part_features, memory_bank):
    raise NotImplementedError("write your pallas kernel here")


# speedup vs baseline: 6.9023x; 6.9023x over previous
import functools

import jax
import jax.numpy as jnp
from jax.experimental import pallas as pl

_TEMPERATURE = 3.0
_TOP_K = 5


def _fused_kernel(feats_ref, mem_ref, soft_ref, sim_ref, *, top_k, inv_temp):
    f = feats_ref[0]
    norm = jnp.sqrt(jnp.sum(f * f, axis=1, keepdims=True))
    f = f / jnp.maximum(norm, 1e-12)

    m = mem_ref[0]
    sim = jax.lax.dot_general(
        f, m, (((1,), (1,)), ((), ())), preferred_element_type=jnp.float32
    )
    sim_ref[0] = sim

    bb, n = sim.shape
    cols = jax.lax.broadcasted_iota(jnp.int32, (bb, n), 1)
    neg_inf = jnp.float32(-jnp.inf)

    work = sim
    sel = jnp.zeros((bb, n), dtype=jnp.bool_)
    row_max = None
    for i in range(top_k):
        mval = jnp.max(work, axis=1, keepdims=True)
        if i == 0:
            row_max = mval
        idx = jnp.min(
            jnp.where(work == mval, cols, jnp.int32(n)), axis=1, keepdims=True
        )
        hit = cols == idx
        sel = jnp.logical_or(sel, hit)
        work = jnp.where(hit, neg_inf, work)

    e = jnp.where(sel, jnp.exp((sim - row_max) * inv_temp), 0.0)
    s = jnp.sum(e, axis=1, keepdims=True)
    soft_ref[0] = e / s


def _build_call(K, B, N, D, block_b):
    kern = functools.partial(
        _fused_kernel, top_k=_TOP_K, inv_temp=1.0 / _TEMPERATURE
    )
    grid = (K, B // block_b)
    return pl.pallas_call(
        kern,
        grid=grid,
        in_specs=[
            pl.BlockSpec((1, block_b, D), lambda k, b: (k, b, 0)),
            pl.BlockSpec((1, N, D), lambda k, b: (k, 0, 0)),
        ],
        out_specs=[
            pl.BlockSpec((1, block_b, N), lambda k, b: (k, b, 0)),
            pl.BlockSpec((1, block_b, N), lambda k, b: (k, b, 0)),
        ],
        out_shape=[
            jax.ShapeDtypeStruct((K, B, N), jnp.float32),
            jax.ShapeDtypeStruct((K, B, N), jnp.float32),
        ],
    )


@jax.jit
def kernel(part_features, memory_bank):
    K, B, D = part_features.shape
    _, N, _ = memory_bank.shape
    block_b = 128 if B % 128 == 0 else B
    soft, sim = _build_call(K, B, N, D, block_b)(part_features, memory_bank)
    return soft, sim


# speedup vs baseline: 15.8497x; 2.2963x over previous
import functools

import jax
import jax.numpy as jnp
from jax.experimental import pallas as pl

_TEMPERATURE = 3.0
_TOP_K = 5


def _fused_kernel(feats_ref, mem_ref, soft_ref, sim_ref, *, top_k, inv_temp):
    f = feats_ref[0]
    norm = jnp.sqrt(jnp.sum(f * f, axis=1, keepdims=True))
    f = f / jnp.maximum(norm, 1e-12)

    m = mem_ref[0]
    sim = jax.lax.dot_general(
        f, m, (((1,), (1,)), ((), ())), preferred_element_type=jnp.float32
    )
    sim_ref[0] = sim

    bb, n = sim.shape
    neg_inf = jnp.float32(-jnp.inf)

    chunk = min(128, n)
    nchunks = n // chunk
    t = [jnp.full((bb, chunk), neg_inf, dtype=jnp.float32) for _ in range(top_k)]
    for c in range(nchunks):
        x = sim[:, c * chunk : (c + 1) * chunk]
        for i in range(top_k):
            hi = jnp.maximum(t[i], x)
            x = jnp.minimum(t[i], x)
            t[i] = hi

    u = jnp.concatenate(t, axis=1)
    m = u.shape[1]
    ucols = jax.lax.broadcasted_iota(jnp.int32, (bb, m), 1)
    vals = []
    for _ in range(top_k):
        mv = jnp.max(u, axis=1, keepdims=True)
        vals.append(mv)
        idx = jnp.min(
            jnp.where(u == mv, ucols, jnp.int32(m)), axis=1, keepdims=True
        )
        u = jnp.where(ucols == idx, neg_inf, u)

    row_max = vals[0]
    thr = vals[top_k - 1]
    s = sum(jnp.exp((v - row_max) * inv_temp) for v in vals)

    e = jnp.exp((sim - row_max) * inv_temp) * (1.0 / s)
    soft_ref[0] = jnp.where(sim >= thr, e, 0.0)


def _build_call(K, B, N, D, block_b):
    kern = functools.partial(
        _fused_kernel, top_k=_TOP_K, inv_temp=1.0 / _TEMPERATURE
    )
    grid = (K, B // block_b)
    return pl.pallas_call(
        kern,
        grid=grid,
        in_specs=[
            pl.BlockSpec((1, block_b, D), lambda k, b: (k, b, 0)),
            pl.BlockSpec((1, N, D), lambda k, b: (k, 0, 0)),
        ],
        out_specs=[
            pl.BlockSpec((1, block_b, N), lambda k, b: (k, b, 0)),
            pl.BlockSpec((1, block_b, N), lambda k, b: (k, b, 0)),
        ],
        out_shape=[
            jax.ShapeDtypeStruct((K, B, N), jnp.float32),
            jax.ShapeDtypeStruct((K, B, N), jnp.float32),
        ],
    )


@jax.jit
def kernel(part_features, memory_bank):
    K, B, D = part_features.shape
    _, N, _ = memory_bank.shape
    block_b = 128 if B % 128 == 0 else B
    soft, sim = _build_call(K, B, N, D, block_b)(part_features, memory_bank)
    return soft, sim


# speedup vs baseline: 18.5118x; 1.1680x over previous
import functools

import jax
import jax.numpy as jnp
from jax.experimental import pallas as pl

_TEMPERATURE = 3.0
_TOP_K = 5


def _fused_kernel(feats_ref, mem_ref, soft_ref, sim_ref, *, top_k, inv_temp):
    f = feats_ref[0]
    norm = jnp.sqrt(jnp.sum(f * f, axis=1, keepdims=True))
    f = f / jnp.maximum(norm, 1e-12)

    m = mem_ref[0]
    sim = jax.lax.dot_general(
        f.astype(jnp.bfloat16), m, (((1,), (1,)), ((), ())),
        preferred_element_type=jnp.float32,
    )
    sim_ref[0] = sim

    bb, n = sim.shape
    neg_inf = jnp.float32(-jnp.inf)
    chunk = min(128, n)
    nchunks = n // chunk
    sub = min(32, bb)

    for r in range(0, bb, sub):
        ssub = sim[r : r + sub, :]

        t = [
            jnp.full((sub, chunk), neg_inf, dtype=jnp.float32)
            for _ in range(top_k)
        ]
        for c in range(nchunks):
            x = ssub[:, c * chunk : (c + 1) * chunk]
            for i in range(top_k):
                hi = jnp.maximum(t[i], x)
                x = jnp.minimum(t[i], x)
                t[i] = hi

        u = jnp.concatenate(t, axis=1)
        m = u.shape[1]
        ucols = jax.lax.broadcasted_iota(jnp.int32, (sub, m), 1)
        vals = []
        for _ in range(top_k):
            mv = jnp.max(u, axis=1, keepdims=True)
            vals.append(mv)
            idx = jnp.min(
                jnp.where(u == mv, ucols, jnp.int32(m)), axis=1, keepdims=True
            )
            u = jnp.where(ucols == idx, neg_inf, u)

        row_max = vals[0]
        thr = vals[top_k - 1]
        s = sum(jnp.exp((v - row_max) * inv_temp) for v in vals)

        e = jnp.exp((ssub - row_max) * inv_temp) * (1.0 / s)
        soft_ref[0, r : r + sub, :] = jnp.where(ssub >= thr, e, 0.0)


def _build_call(K, B, N, D, block_b):
    kern = functools.partial(
        _fused_kernel, top_k=_TOP_K, inv_temp=1.0 / _TEMPERATURE
    )
    grid = (K, B // block_b)
    return pl.pallas_call(
        kern,
        grid=grid,
        in_specs=[
            pl.BlockSpec((1, block_b, D), lambda k, b: (k, b, 0)),
            pl.BlockSpec((1, N, D), lambda k, b: (k, 0, 0)),
        ],
        out_specs=[
            pl.BlockSpec((1, block_b, N), lambda k, b: (k, b, 0)),
            pl.BlockSpec((1, block_b, N), lambda k, b: (k, b, 0)),
        ],
        out_shape=[
            jax.ShapeDtypeStruct((K, B, N), jnp.float32),
            jax.ShapeDtypeStruct((K, B, N), jnp.float32),
        ],
    )


@jax.jit
def kernel(part_features, memory_bank):
    K, B, D = part_features.shape
    _, N, _ = memory_bank.shape
    block_b = 256 if B % 256 == 0 else B
    soft, sim = _build_call(K, B, N, D, block_b)(
        part_features, memory_bank.astype(jnp.bfloat16)
    )
    return soft, sim


# speedup vs baseline: 20.7867x; 1.1229x over previous
import functools

import jax
import jax.numpy as jnp
from jax.experimental import pallas as pl

_TEMPERATURE = 3.0
_TOP_K = 5


def _fused_kernel(feats_ref, mem_ref, soft_ref, sim_ref, *, top_k, inv_temp):
    f = feats_ref[0]
    norm = jnp.sqrt(jnp.sum(f * f, axis=1, keepdims=True))
    f = f / jnp.maximum(norm, 1e-12)

    m = mem_ref[0]
    sim = jax.lax.dot_general(
        f.astype(jnp.bfloat16), m, (((1,), (1,)), ((), ())),
        preferred_element_type=jnp.float32,
    )
    sim_ref[0] = sim

    bb, n = sim.shape
    neg_inf = jnp.float32(-jnp.inf)
    chunk = min(128, n)
    nchunks = n // chunk
    sub = min(32, bb)

    for r in range(0, bb, sub):
        ssub = sim[r : r + sub, :]

        t = [
            jnp.full((sub, chunk), neg_inf, dtype=jnp.float32)
            for _ in range(top_k)
        ]
        for c in range(nchunks):
            x = ssub[:, c * chunk : (c + 1) * chunk]
            for i in range(top_k):
                hi = jnp.maximum(t[i], x)
                x = jnp.minimum(t[i], x)
                t[i] = hi

        u = jnp.concatenate(t, axis=1)
        vals = []
        for _ in range(top_k):
            mv = jnp.max(u, axis=1, keepdims=True)
            vals.append(mv)
            u = jnp.where(u == mv, neg_inf, u)

        row_max = vals[0]
        thr = vals[top_k - 1]
        log2e = jnp.float32(1.4426950408889634)
        a = jnp.float32(inv_temp) * log2e
        s = sum(jnp.exp2((v - row_max) * a) for v in vals)
        b = -row_max * a - jnp.log2(s)

        e = jnp.exp2(ssub * a + b)
        soft_ref[0, r : r + sub, :] = jnp.where(ssub >= thr, e, 0.0)


def _build_call(K, B, N, D, block_b):
    kern = functools.partial(
        _fused_kernel, top_k=_TOP_K, inv_temp=1.0 / _TEMPERATURE
    )
    grid = (K, B // block_b)
    return pl.pallas_call(
        kern,
        grid=grid,
        in_specs=[
            pl.BlockSpec((1, block_b, D), lambda k, b: (k, b, 0)),
            pl.BlockSpec((1, N, D), lambda k, b: (k, 0, 0)),
        ],
        out_specs=[
            pl.BlockSpec((1, block_b, N), lambda k, b: (k, b, 0)),
            pl.BlockSpec((1, block_b, N), lambda k, b: (k, b, 0)),
        ],
        out_shape=[
            jax.ShapeDtypeStruct((K, B, N), jnp.float32),
            jax.ShapeDtypeStruct((K, B, N), jnp.float32),
        ],
    )


@jax.jit
def kernel(part_features, memory_bank):
    K, B, D = part_features.shape
    _, N, _ = memory_bank.shape
    block_b = 256 if B % 256 == 0 else B
    soft, sim = _build_call(K, B, N, D, block_b)(
        part_features, memory_bank.astype(jnp.bfloat16)
    )
    return soft, sim
